# baseline (speedup 1.0000x reference)
import functools

import jax
import jax.numpy as jnp
from jax import lax
from jax.experimental import pallas as pl
from jax.experimental.pallas import tpu as pltpu
from jax.experimental.pallas import tpu_sc as plsc

B = 100
A = 100
DEG = 32
SLOTS = DEG + 1
D = 128
NPAIR = B * A
NUNITS = B * SLOTS
IDXW = 112
EWIN = 160

_info = plsc.get_sparse_core_info()
NW = _info.num_cores * _info.num_subcores

NBUF = 5
NPAD = NBUF * (-(-(-(-NUNITS // NW)) // NBUF))


@functools.partial(
    pl.kernel,
    out_type=jax.ShapeDtypeStruct((B, SLOTS, A, D), jnp.float32),
    mesh=plsc.VectorSubcoreMesh(core_axis_name="c", subcore_axis_name="s"),
    compiler_params=pltpu.CompilerParams(needs_layout_passes=False),
    scratch_types=[
        pltpu.VMEM((EWIN, A), jnp.int32),
        pltpu.VMEM((NBUF, IDXW), jnp.int32),
        pltpu.VMEM((A, D), jnp.float32),
        pltpu.VMEM((A, D), jnp.float32),
        pltpu.VMEM((A, D), jnp.float32),
        pltpu.VMEM((A, D), jnp.float32),
        pltpu.VMEM((A, D), jnp.float32),
        pltpu.SemaphoreType.DMA,
        pltpu.SemaphoreType.DMA,
        pltpu.SemaphoreType.DMA,
        pltpu.SemaphoreType.DMA,
        pltpu.SemaphoreType.DMA,
        pltpu.SemaphoreType.DMA,
        pltpu.SemaphoreType.DMA,
        pltpu.SemaphoreType.DMA,
        pltpu.SemaphoreType.DMA,
        pltpu.SemaphoreType.DMA,
    ],
)
def _graph_gather(atoms_hbm, edges_hbm, out_hbm, e_v, idx_v,
                  rows_v0, rows_v1, rows_v2, rows_v3, rows_v4,
                  gsem0, gsem1, gsem2, gsem3, gsem4,
                  osem0, osem1, osem2, osem3, osem4):
    rows_vs = (rows_v0, rows_v1, rows_v2, rows_v3, rows_v4)
    gsems = (gsem0, gsem1, gsem2, gsem3, gsem4)
    osems = (osem0, osem1, osem2, osem3, osem4)
    wid = lax.axis_index("s") * _info.num_cores + lax.axis_index("c")
    c0 = wid * NUNITS // NW
    c1 = (wid + 1) * NUNITS // NW

    ebase = pl.multiple_of(
        jnp.minimum((c0 // SLOTS) * DEG, B * DEG - EWIN), 8)
    pltpu.sync_copy(edges_hbm.at[pl.ds(ebase, EWIN), :], e_v)

    def unit_of(j):
        return jnp.minimum(c0 + j, c1 - 1)

    def stage_gather(j, b):
        u = unit_of(j)
        ub = u // SLOTS
        slot = u - ub * SLOTS
        row0 = ub * A
        erow = jnp.maximum(ub * DEG + slot - 1 - ebase, 0)

        @pl.when(slot == 0)
        def _self():
            def body(k, carry):
                a = jnp.minimum(k * 16 + lax.iota(jnp.int32, 16), A - 1)
                idx_v[b, pl.ds(k * 16, 16)] = row0 + a
                return carry
            lax.fori_loop(0, IDXW // 16, body, 0)
            pltpu.async_copy(
                atoms_hbm.at[idx_v.at[b, pl.ds(0, A)]], rows_vs[b], gsems[b])

        @pl.when(slot != 0)
        def _neigh():
            erow16 = jnp.full((16,), 0, jnp.int32) + erow
            def body(k, carry):
                a = jnp.minimum(k * 16 + lax.iota(jnp.int32, 16), A - 1)
                ev = plsc.load_gather(e_v, [erow16, a])
                idx_v[b, pl.ds(k * 16, 16)] = row0 + ev
                return carry
            lax.fori_loop(0, IDXW // 16, body, 0)
            pltpu.async_copy(
                atoms_hbm.at[idx_v.at[b, pl.ds(0, A)]], rows_vs[b], gsems[b])

    def wait_gather(b):
        pltpu.make_async_copy(
            atoms_hbm.at[idx_v.at[b, pl.ds(0, A)]], rows_vs[b], gsems[b]).wait()

    def start_out(j, b):
        u = unit_of(j)
        ub = u // SLOTS
        slot = u - ub * SLOTS
        pltpu.async_copy(rows_vs[b], out_hbm.at[ub, slot], osems[b])

    def wait_out(b):
        pltpu.make_async_copy(rows_vs[b], out_hbm.at[0, 0], osems[b]).wait()

    stage_gather(0, 0)
    stage_gather(1, 1)
    stage_gather(2, 2)
    stage_gather(3, 3)
    wait_gather(0)
    start_out(0, 0)
    stage_gather(4, 4)
    wait_gather(1)
    start_out(1, 1)

    def group_body(g, carry):
        for b in range(NBUF):
            j = NBUF * g + b
            wait_out(b)
            stage_gather(j, b)
            b3 = (b + 2) % NBUF
            wait_gather(b3)
            start_out(j - 3, b3)
        return carry

    lax.fori_loop(1, NPAD // NBUF, group_body, 0)
    wait_gather(2)
    start_out(NPAD - 3, 2)
    wait_gather(3)
    start_out(NPAD - 2, 3)
    wait_gather(4)
    start_out(NPAD - 1, 4)
    for b in range(NBUF):
        wait_out(b)


def kernel(atoms, edges):
    assert atoms.shape == (B, A, D) and edges.shape == (B, A, DEG)
    ef = edges.transpose(0, 2, 1).reshape(B * DEG, A)
    out4 = _graph_gather(atoms.reshape(NPAIR, D), ef)
    return out4.transpose(0, 2, 1, 3)


# speedup vs baseline: 1.0136x; 1.0136x over previous
import functools

import jax
import jax.numpy as jnp
from jax import lax
from jax.experimental import pallas as pl
from jax.experimental.pallas import tpu as pltpu
from jax.experimental.pallas import tpu_sc as plsc

B = 100
A = 100
DEG = 32
SLOTS = DEG + 1
D = 128
NPAIR = B * A
NUNITS = B * SLOTS
IDXW = 112
EWIN = 160

_info = plsc.get_sparse_core_info()
NW = _info.num_cores * _info.num_subcores

NBUF = 4
NPAD = NBUF * (-(-(-(-NUNITS // NW)) // NBUF))


@functools.partial(
    pl.kernel,
    out_type=jax.ShapeDtypeStruct((B, SLOTS, A, D), jnp.float32),
    mesh=plsc.VectorSubcoreMesh(core_axis_name="c", subcore_axis_name="s"),
    compiler_params=pltpu.CompilerParams(needs_layout_passes=False),
    scratch_types=[
        pltpu.VMEM((EWIN, A), jnp.int32),
        pltpu.VMEM((NBUF, IDXW), jnp.int32),
        pltpu.VMEM((A, D), jnp.float32),
        pltpu.VMEM((A, D), jnp.float32),
        pltpu.VMEM((A, D), jnp.float32),
        pltpu.VMEM((A, D), jnp.float32),
        pltpu.SemaphoreType.DMA,
        pltpu.SemaphoreType.DMA,
        pltpu.SemaphoreType.DMA,
        pltpu.SemaphoreType.DMA,
        pltpu.SemaphoreType.DMA,
        pltpu.SemaphoreType.DMA,
        pltpu.SemaphoreType.DMA,
        pltpu.SemaphoreType.DMA,
    ],
)
def _graph_gather(atoms_hbm, edges_hbm, out_hbm, e_v, idx_v,
                  rows_v0, rows_v1, rows_v2, rows_v3,
                  gsem0, gsem1, gsem2, gsem3, osem0, osem1, osem2, osem3):
    rows_vs = (rows_v0, rows_v1, rows_v2, rows_v3)
    gsems = (gsem0, gsem1, gsem2, gsem3)
    osems = (osem0, osem1, osem2, osem3)
    wid = lax.axis_index("s") * _info.num_cores + lax.axis_index("c")
    c0 = wid * NUNITS // NW
    c1 = (wid + 1) * NUNITS // NW

    ebase = pl.multiple_of(
        jnp.minimum((c0 // SLOTS) * DEG, B * DEG - EWIN), 8)
    pltpu.sync_copy(edges_hbm.at[pl.ds(ebase, EWIN), :], e_v)

    def unit_of(j):
        return jnp.minimum(c0 + j, c1 - 1)

    def stage_gather(j, b):
        u = unit_of(j)
        ub = u // SLOTS
        slot = u - ub * SLOTS
        row0 = ub * A
        erow = jnp.maximum(ub * DEG + slot - 1 - ebase, 0)

        @pl.when(slot == 0)
        def _self():
            def body(k, carry):
                a = jnp.minimum(k * 16 + lax.iota(jnp.int32, 16), A - 1)
                idx_v[b, pl.ds(k * 16, 16)] = row0 + a
                return carry
            lax.fori_loop(0, IDXW // 16, body, 0)
            pltpu.async_copy(
                atoms_hbm.at[idx_v.at[b, pl.ds(0, A)]], rows_vs[b], gsems[b])

        @pl.when(slot != 0)
        def _neigh():
            erow16 = jnp.full((16,), 0, jnp.int32) + erow
            def body(k, carry):
                a = jnp.minimum(k * 16 + lax.iota(jnp.int32, 16), A - 1)
                ev = plsc.load_gather(e_v, [erow16, a])
                idx_v[b, pl.ds(k * 16, 16)] = row0 + ev
                return carry
            lax.fori_loop(0, IDXW // 16, body, 0)
            pltpu.async_copy(
                atoms_hbm.at[idx_v.at[b, pl.ds(0, A)]], rows_vs[b], gsems[b])

    def wait_gather(b):
        pltpu.make_async_copy(
            atoms_hbm.at[idx_v.at[b, pl.ds(0, A)]], rows_vs[b], gsems[b]).wait()

    def start_out(j, b):
        u = unit_of(j)
        ub = u // SLOTS
        slot = u - ub * SLOTS
        pltpu.async_copy(rows_vs[b], out_hbm.at[ub, slot], osems[b])

    def wait_out(b):
        pltpu.make_async_copy(rows_vs[b], out_hbm.at[0, 0], osems[b]).wait()

    stage_gather(0, 0)
    stage_gather(1, 1)
    stage_gather(2, 2)
    wait_gather(0)
    start_out(0, 0)
    stage_gather(3, 3)
    wait_gather(1)
    start_out(1, 1)

    def group_body(g, carry):
        for b in range(NBUF):
            j = NBUF * g + b
            wait_out(b)
            stage_gather(j, b)
            b2 = (b + 2) % NBUF
            wait_gather(b2)
            start_out(j - 2, b2)
        return carry

    lax.fori_loop(1, NPAD // NBUF, group_body, 0)
    wait_gather(2)
    start_out(NPAD - 2, 2)
    wait_gather(3)
    start_out(NPAD - 1, 3)
    for b in range(NBUF):
        wait_out(b)


def kernel(atoms, edges):
    assert atoms.shape == (B, A, D) and edges.shape == (B, A, DEG)
    ef = edges.transpose(0, 2, 1).reshape(B * DEG, A)
    out4 = _graph_gather(atoms.reshape(NPAIR, D), ef)
    return out4.transpose(0, 2, 1, 3)


# speedup vs baseline: 1.0169x; 1.0033x over previous
import functools

import jax
import jax.numpy as jnp
from jax import lax
from jax.experimental import pallas as pl
from jax.experimental.pallas import tpu as pltpu
from jax.experimental.pallas import tpu_sc as plsc

B = 100
A = 100
DEG = 32
SLOTS = DEG + 1
D = 128
NPAIR = B * A
NUNITS = B * SLOTS
IDXW = 112
EWIN = 160

_info = plsc.get_sparse_core_info()
NW = _info.num_cores * _info.num_subcores

NBUF = 4
NPAD = NBUF * (-(-(-(-NUNITS // NW)) // NBUF))


@functools.partial(
    pl.kernel,
    out_type=jax.ShapeDtypeStruct((B, SLOTS, A, D), jnp.float32),
    mesh=plsc.VectorSubcoreMesh(core_axis_name="c", subcore_axis_name="s"),
    compiler_params=pltpu.CompilerParams(needs_layout_passes=False),
    scratch_types=[
        pltpu.VMEM((EWIN, A), jnp.int32),
        pltpu.VMEM((IDXW,), jnp.int32),
        pltpu.VMEM((A, D), jnp.float32),
        pltpu.VMEM((A, D), jnp.float32),
        pltpu.VMEM((A, D), jnp.float32),
        pltpu.VMEM((A, D), jnp.float32),
        pltpu.SemaphoreType.DMA,
        pltpu.SemaphoreType.DMA,
        pltpu.SemaphoreType.DMA,
        pltpu.SemaphoreType.DMA,
        pltpu.SemaphoreType.DMA,
        pltpu.SemaphoreType.DMA,
        pltpu.SemaphoreType.DMA,
        pltpu.SemaphoreType.DMA,
    ],
)
def _graph_gather(atoms_hbm, edges_hbm, out_hbm, e_v, iota_v,
                  rows_v0, rows_v1, rows_v2, rows_v3,
                  gsem0, gsem1, gsem2, gsem3, osem0, osem1, osem2, osem3):
    rows_vs = (rows_v0, rows_v1, rows_v2, rows_v3)
    gsems = (gsem0, gsem1, gsem2, gsem3)
    osems = (osem0, osem1, osem2, osem3)
    wid = lax.axis_index("s") * _info.num_cores + lax.axis_index("c")
    c0 = wid * NUNITS // NW
    c1 = (wid + 1) * NUNITS // NW

    ebase = pl.multiple_of(
        jnp.minimum((c0 // SLOTS) * DEG, B * DEG - EWIN), 8)
    pltpu.sync_copy(edges_hbm.at[pl.ds(ebase, EWIN), :], e_v)
    for k in range(IDXW // 16):
        iota_v[pl.ds(k * 16, 16)] = k * 16 + lax.iota(jnp.int32, 16)

    def unit_of(j):
        return jnp.minimum(c0 + j, c1 - 1)

    def stage_gather(j, b):
        u = unit_of(j)
        ub = u // SLOTS
        slot = u - ub * SLOTS
        erow = jnp.maximum(ub * DEG + slot - 1 - ebase, 0)

        @pl.when(slot == 0)
        def _self():
            pltpu.async_copy(
                atoms_hbm.at[ub].at[iota_v.at[pl.ds(0, A)]],
                rows_vs[b], gsems[b])

        @pl.when(slot != 0)
        def _neigh():
            pltpu.async_copy(
                atoms_hbm.at[ub].at[e_v.at[erow]], rows_vs[b], gsems[b])

    def wait_gather(b):
        pltpu.make_async_copy(
            atoms_hbm.at[0].at[iota_v.at[pl.ds(0, A)]],
            rows_vs[b], gsems[b]).wait()

    def start_out(j, b):
        u = unit_of(j)
        ub = u // SLOTS
        slot = u - ub * SLOTS
        pltpu.async_copy(rows_vs[b], out_hbm.at[ub, slot], osems[b])

    def wait_out(b):
        pltpu.make_async_copy(rows_vs[b], out_hbm.at[0, 0], osems[b]).wait()

    stage_gather(0, 0)
    stage_gather(1, 1)
    stage_gather(2, 2)
    wait_gather(0)
    start_out(0, 0)
    stage_gather(3, 3)
    wait_gather(1)
    start_out(1, 1)

    def group_body(g, carry):
        for b in range(NBUF):
            j = NBUF * g + b
            wait_out(b)
            stage_gather(j, b)
            b2 = (b + 2) % NBUF
            wait_gather(b2)
            start_out(j - 2, b2)
        return carry

    lax.fori_loop(1, NPAD // NBUF, group_body, 0)
    wait_gather(2)
    start_out(NPAD - 2, 2)
    wait_gather(3)
    start_out(NPAD - 1, 3)
    for b in range(NBUF):
        wait_out(b)


def kernel(atoms, edges):
    assert atoms.shape == (B, A, D) and edges.shape == (B, A, DEG)
    ef = edges.transpose(0, 2, 1).reshape(B * DEG, A)
    out4 = _graph_gather(atoms, ef)
    return out4.transpose(0, 2, 1, 3)
